# Initial kernel scaffold; baseline (speedup 1.0000x reference)
import jax
import jax.numpy as jnp
from jax.experimental import pallas as pl


def kernel(edge_index, edge_vals, x):
    raise NotImplementedError("write your pallas kernel here")


# speedup vs baseline: 4.4279x; 4.4279x over previous
import functools

import jax
import jax.numpy as jnp
from jax import lax
from jax.experimental import pallas as pl
from jax.experimental.pallas import tpu as pltpu
from jax.experimental.pallas import tpu_sc as plsc

N_NODES = 10000
N_EDGES = 320000
D = 128
NC = 2
NS = 16
NW = NC * NS
E_PER_TILE = N_EDGES // NW
G = 80
NCHUNK = E_PER_TILE // G
N_PAD = 10240
ROWS_PER_TILE = N_PAD // NS
ZR = 128


def _sc_partials(rows, cols, edge_vals, x):
    mesh = plsc.VectorSubcoreMesh(core_axis_name="c", subcore_axis_name="s")

    @functools.partial(
        pl.kernel,
        mesh=mesh,
        out_type=jax.ShapeDtypeStruct((NC, N_PAD, D), jnp.float32),
        scratch_types=[
            pltpu.VMEM((1, G), jnp.int32),
            pltpu.VMEM((1, G), jnp.int32),
            pltpu.VMEM((G,), jnp.float32),
            pltpu.VMEM((G, D), jnp.float32),
            pltpu.VMEM((ZR, D), jnp.float32),
            pltpu.VMEM_SHARED((N_PAD, D), jnp.float32),
            pltpu.SemaphoreType.DMA,
        ],
    )
    def body(rows_hbm, cols_hbm, ev_hbm, x_hbm, out_hbm, cols_v, rows_v,
             vals_v, gbuf, zbuf, acc, sem):
        cid = lax.axis_index("c")
        sid = lax.axis_index("s")
        w = cid * NS + sid

        zrow = jnp.zeros((16,), jnp.float32)

        def zinit(i, carry):
            for j in range(D // 16):
                zbuf[i, pl.ds(j * 16, 16)] = zrow
            return carry

        lax.fori_loop(0, ZR, zinit, 0)
        for k in range(ROWS_PER_TILE // ZR):
            pltpu.sync_copy(
                zbuf, acc.at[pl.ds(sid * ROWS_PER_TILE + k * ZR, ZR)])
        plsc.subcore_barrier()

        base = w * E_PER_TILE

        def chunk(i, carry):
            e0 = base + i * G
            pltpu.sync_copy(cols_hbm.at[pl.ds(e0, G)], cols_v.at[0])
            pltpu.sync_copy(rows_hbm.at[pl.ds(e0, G)], rows_v.at[0])
            pltpu.sync_copy(ev_hbm.at[pl.ds(e0, G)], vals_v)
            pltpu.async_copy(x_hbm.at[cols_v.at[0]], gbuf, sem).wait()

            def scale(g, c2):
                v16 = vals_v[pl.ds(g * 16, 16)]
                for l in range(16):
                    s = v16[l]
                    e = g * 16 + l
                    for j in range(D // 16):
                        gbuf[e, pl.ds(j * 16, 16)] = (
                            gbuf[e, pl.ds(j * 16, 16)] * s)
                return c2

            lax.fori_loop(0, G // 16, scale, 0)
            pltpu.sync_copy(gbuf, acc.at[rows_v.at[0]], add=True)
            return carry

        lax.fori_loop(0, NCHUNK, chunk, 0)
        plsc.subcore_barrier()

        pltpu.sync_copy(
            acc.at[pl.ds(sid * ROWS_PER_TILE, ROWS_PER_TILE)],
            out_hbm.at[cid, pl.ds(sid * ROWS_PER_TILE, ROWS_PER_TILE)])

    return body(rows, cols, edge_vals, x)


def _tc_add(partials):
    def body(a_ref, b_ref, o_ref):
        o_ref[...] = a_ref[...] + b_ref[...]

    return pl.pallas_call(
        body,
        grid=(10,),
        in_specs=[
            pl.BlockSpec((N_PAD // 10, D), lambda i: (i, 0)),
            pl.BlockSpec((N_PAD // 10, D), lambda i: (i, 0)),
        ],
        out_specs=pl.BlockSpec((N_PAD // 10, D), lambda i: (i, 0)),
        out_shape=jax.ShapeDtypeStruct((N_PAD, D), jnp.float32),
    )(partials[0], partials[1])


def kernel(edge_index, edge_vals, x):
    partials = _sc_partials(edge_index[0], edge_index[1], edge_vals, x)
    return _tc_add(partials)[:N_NODES]
